# baseline (speedup 1.0000x reference)
import jax
import jax.numpy as jnp
from jax import lax
from jax.experimental import pallas as pl
from jax.experimental.pallas import tpu as pltpu
from jax.experimental.pallas import tpu_sc as plsc

N = 5000
LANES = 16
NPAD = 5120
NCH = NPAD // LANES
NCLS = 80
HUMAN_IDX = 1
NMS_THRESH = 0.5
SCORE_THRESH = 0.2
KCAP = 15
TILES = 16
CPT = NCLS // TILES
NEGS = -3.0e38
DUMMY = 3.0e9
BIGI = 2**30


def _nms_body(bfh, sch, lbh, obh, osh, olh,
              vbf, vsc, vlb, rmidx, rmlab, midx, ms,
              kvs, kvi, gsc, gidx, heads_s, heads_i, ptrv, rb, rs, rl,
              sem1, sem2, sem3, ssc, sidx):
    core = lax.axis_index("c")
    sub = lax.axis_index("s")
    lanes = lax.iota(jnp.int32, LANES)
    ones = lanes >= 0
    negs16 = jnp.full((LANES,), NEGS, jnp.float32)
    bigi16 = jnp.full((LANES,), BIGI, jnp.int32)

    @pl.when(core == 0)
    def _():
        cp1 = pltpu.async_copy(bfh, vbf, sem1)
        cp2 = pltpu.async_copy(sch, vsc.at[pl.ds(0, N)], sem2)
        cp3 = pltpu.async_copy(lbh, vlb.at[pl.ds(0, N)], sem3)
        with jax.named_scope("ph_dma"):
            cp2.wait()
            cp3.wait()
        negone = jnp.full((LANES,), -1.0, jnp.float32)
        negonei = jnp.full((LANES,), -1, jnp.int32)
        for k in range(8):
            plsc.store_compressed(vsc.at[pl.ds(N + 16 * k, LANES)], negone,
                                  mask=ones)
            plsc.store_compressed(vlb.at[pl.ds(N + 16 * k, LANES)], negonei,
                                  mask=ones)
        vsc[pl.ds(NPAD, LANES)] = negone
        vlb[pl.ds(NPAD, LANES)] = negonei

        lo = sub * CPT

        def rscan(j, cnt):
            lab16 = vlb[pl.ds(2 * j * LANES, LANES)]
            sc16 = vsc[pl.ds(2 * j * LANES, LANES)]
            m = (lab16 >= lo) & (lab16 < lo + CPT) & (sc16 >= SCORE_THRESH)
            idx16 = 2 * j * LANES + lanes
            plsc.store_compressed(rmidx.at[pl.ds(cnt, LANES)], idx16, mask=m)
            plsc.store_compressed(rmlab.at[pl.ds(cnt, LANES)], lab16, mask=m)
            cnt = cnt + plsc.all_reduce_population_count(m)[0]
            lab16 = vlb[pl.ds((2 * j + 1) * LANES, LANES)]
            sc16 = vsc[pl.ds((2 * j + 1) * LANES, LANES)]
            m = (lab16 >= lo) & (lab16 < lo + CPT) & (sc16 >= SCORE_THRESH)
            idx16 = (2 * j + 1) * LANES + lanes
            plsc.store_compressed(rmidx.at[pl.ds(cnt, LANES)], idx16, mask=m)
            plsc.store_compressed(rmlab.at[pl.ds(cnt, LANES)], lab16, mask=m)
            return cnt + plsc.all_reduce_population_count(m)[0]

        with jax.named_scope("ph_rscan"):
            rcnt = lax.fori_loop(0, NCH // 2, rscan, jnp.int32(0))
        plsc.store_compressed(rmlab.at[pl.ds(rcnt, LANES)],
                              jnp.full((LANES,), -1, jnp.int32), mask=ones)
        rch = (rcnt + (LANES - 1)) >> 4

        def mx_body(j, acc):
            a = jnp.maximum(vbf[pl.ds(j * 2 * LANES, LANES)],
                            vbf[pl.ds(j * 2 * LANES + LANES, LANES)])
            return jnp.maximum(acc, a)

        with jax.named_scope("ph_maxc"):
            cp1.wait()
            acc = lax.fori_loop(0, (4 * N) // (2 * LANES), mx_body, negs16)
        maxc = jnp.max(acc) + jnp.float32(1.0)

        for k in range(CPT):
            c = lo + k
            off = c.astype(jnp.float32) * maxc

            def scan_body(j, cnt):
                lab16 = rmlab[pl.ds(j * LANES, LANES)]
                m = lab16 == c
                plsc.store_compressed(midx.at[pl.ds(cnt, LANES)],
                                      rmidx[pl.ds(j * LANES, LANES)], mask=m)
                return cnt + plsc.all_reduce_population_count(m)[0]

            with jax.named_scope("ph_l2scan"):
                cnt = lax.fori_loop(0, rch, scan_body, jnp.int32(0))
                nchk0 = (cnt + (LANES - 1)) >> 4

                def ms_body(j, _):
                    mi = midx[pl.ds(j * LANES, LANES)]
                    ms[pl.ds(j * LANES, LANES)] = plsc.load_gather(vsc, [mi])
                    return 0

                plsc.store_compressed(midx.at[pl.ds(cnt, LANES)],
                                      jnp.zeros((LANES,), jnp.int32),
                                      mask=ones)
                lax.fori_loop(0, nchk0, ms_body, 0)
            plsc.store_compressed(ms.at[pl.ds(cnt, LANES)], negs16, mask=ones)

            def cond(st):
                return (st[0] < cnt) & (st[1] < KCAP)

            def body(st):
                nproc, kcnt, kx1, ky1, kx2, ky2, kid, ksc = st
                nchk = (cnt + (LANES - 1)) >> 4

                def am_body(j, s):
                    bv, bp = s
                    v = ms[pl.ds(j * LANES, LANES)]
                    upd = v > bv
                    return jnp.where(upd, v, bv), jnp.where(upd, j, bp)

                bv, bp = lax.fori_loop(0, nchk, am_body,
                                       (negs16, jnp.zeros((LANES,), jnp.int32)))
                gmax = jnp.max(bv)
                posl = jnp.where(bv == gmax, bp * LANES + lanes, BIGI)
                pos = jnp.min(posl)
                posv = jnp.full((LANES,), pos, jnp.int32)
                plsc.store_scatter(ms, [posv], negs16, mask=lanes == 0)
                giv = plsc.load_gather(midx, [posv])
                g4 = giv * 4
                cx1 = plsc.load_gather(vbf, [g4]) + off
                cy1 = plsc.load_gather(vbf, [g4 + 1]) + off
                cx2 = plsc.load_gather(vbf, [g4 + 2]) + off
                cy2 = plsc.load_gather(vbf, [g4 + 3]) + off
                w = jnp.maximum(jnp.minimum(kx2, cx2) - jnp.maximum(kx1, cx1), 0.0)
                h = jnp.maximum(jnp.minimum(ky2, cy2) - jnp.maximum(ky1, cy1), 0.0)
                inter = w * h
                ka = (kx2 - kx1) * (ky2 - ky1)
                ca = (cx2 - cx1) * (cy2 - cy1)
                iou = inter / jnp.maximum(ka + ca - inter, jnp.float32(1e-9))
                sup = plsc.all_reduce_population_count(iou > NMS_THRESH)[0] > 0
                addm = jnp.logical_and(jnp.logical_not(sup), lanes == kcnt)
                kx1 = jnp.where(addm, cx1, kx1)
                ky1 = jnp.where(addm, cy1, ky1)
                kx2 = jnp.where(addm, cx2, kx2)
                ky2 = jnp.where(addm, cy2, ky2)
                kid = jnp.where(addm, giv, kid)
                ksc = jnp.where(addm, gmax, ksc)
                kcnt = kcnt + jnp.where(sup, 0, 1).astype(jnp.int32)
                return (nproc + 1, kcnt, kx1, ky1, kx2, ky2, kid, ksc)

            dummy16 = jnp.full((LANES,), DUMMY, jnp.float32)
            with jax.named_scope("ph_nms"):
                st = lax.while_loop(cond, body,
                                    (jnp.int32(0), jnp.int32(0),
                                     dummy16, dummy16, dummy16, dummy16,
                                     bigi16, negs16))
            kvs[pl.ds(k * LANES, LANES)] = st[7]
            kvi[pl.ds(k * LANES, LANES)] = st[6]

        pltpu.sync_copy(kvs, ssc.at[pl.ds(lo * LANES, CPT * LANES)])
        pltpu.sync_copy(kvi, sidx.at[pl.ds(lo * LANES, CPT * LANES)])

        plsc.subcore_barrier()

        @pl.when(sub == 0)
        def _():
          with jax.named_scope("ph_merge"):
            pltpu.sync_copy(ssc, gsc)
            pltpu.sync_copy(sidx, gidx)
            hs = gsc[pl.ds(HUMAN_IDX * LANES, LANES)]
            hi = gidx[pl.ds(HUMAN_IDX * LANES, LANES)]
            gsc[pl.ds(HUMAN_IDX * LANES, LANES)] = negs16
            for j in range(NCLS // LANES):
                rowv = (j * LANES + lanes) * LANES
                heads_s[pl.ds(j * LANES, LANES)] = plsc.load_gather(gsc, [rowv])
                heads_i[pl.ds(j * LANES, LANES)] = plsc.load_gather(gidx, [rowv])
            one16 = jnp.full((LANES,), 1, jnp.int32)
            for j in range(NCLS // LANES):
                ptrv[pl.ds(j * LANES, LANES)] = one16

            def ext_body(t, s):
                osc, oidx = s

                def hb(j, hst):
                    bv, bi, bp = hst
                    v = heads_s[pl.ds(j * LANES, LANES)]
                    iv = heads_i[pl.ds(j * LANES, LANES)]
                    upd = (v > bv) | ((v == bv) & (iv < bi))
                    return (jnp.where(upd, v, bv), jnp.where(upd, iv, bi),
                            jnp.where(upd, j, bp))

                bv, bi, bp = lax.fori_loop(0, NCLS // LANES, hb,
                                           (negs16, bigi16,
                                            jnp.zeros((LANES,), jnp.int32)))
                gmax = jnp.max(bv)
                gidw = jnp.min(jnp.where(bv == gmax, bi, BIGI))
                cls = jnp.min(jnp.where((bv == gmax) & (bi == gidw),
                                        bp * LANES + lanes, BIGI))
                clsv = jnp.full((LANES,), cls, jnp.int32)
                p = plsc.load_gather(ptrv, [clsv])
                plsc.store_scatter(ptrv, [clsv], p + 1, mask=lanes == 0)
                psafe = jnp.minimum(p, LANES - 1)
                hv = plsc.load_gather(gsc, [clsv * LANES + psafe])
                hiv = plsc.load_gather(gidx, [clsv * LANES + psafe])
                hv = jnp.where(p >= LANES, negs16, hv)
                plsc.store_scatter(heads_s, [clsv], hv, mask=lanes == 0)
                plsc.store_scatter(heads_i, [clsv], hiv, mask=lanes == 0)
                valid = gmax > jnp.float32(-1.0e37)
                osc = jnp.where((lanes == t) & valid, gmax, osc)
                oidx = jnp.where((lanes == t) & valid, gidw, oidx)
                return (osc, oidx)

            osc, oidx = lax.fori_loop(0, KCAP, ext_body, (negs16, bigi16))

            heads_s[pl.ds(0, LANES)] = hs
            heads_i[pl.ds(0, LANES)] = hi
            heads_s[pl.ds(LANES, LANES)] = osc
            heads_i[pl.ds(LANES, LANES)] = oidx

            zf16 = jnp.zeros((LANES,), jnp.float32)
            for j in range(8):
                rb[pl.ds(j * LANES, LANES)] = zf16
            rs[pl.ds(0, LANES)] = zf16
            rs[pl.ds(LANES, LANES)] = zf16
            neg1 = jnp.full((LANES,), -1, jnp.int32)
            rl[pl.ds(0, LANES)] = neg1
            rl[pl.ds(LANES, LANES)] = neg1

            def mg_body(t, s):
                hp, op = s
                hpv = jnp.full((LANES,), hp, jnp.int32)
                opv = jnp.full((LANES,), op + LANES, jnp.int32)
                hv = plsc.load_gather(heads_s, [hpv])
                hiv = plsc.load_gather(heads_i, [hpv])
                ov = plsc.load_gather(heads_s, [opv])
                oiv = plsc.load_gather(heads_i, [opv])
                hvs = hv[0]
                ovs = ov[0]
                his = hiv[0]
                ois = oiv[0]
                takeh = (hvs > ovs) | ((hvs == ovs) & (his < ois))
                cs = jnp.where(takeh, hv, ov)
                ci = jnp.where(takeh, hiv, oiv)
                valid = cs[0] > jnp.float32(-1.0e37)
                cis = jnp.where(valid, ci, jnp.zeros((LANES,), jnp.int32))
                ci4 = cis * 4
                m0 = (lanes == 0) & valid
                tv = jnp.full((LANES,), t, jnp.int32)
                bx1 = plsc.load_gather(vbf, [ci4])
                by1 = plsc.load_gather(vbf, [ci4 + 1])
                bx2 = plsc.load_gather(vbf, [ci4 + 2])
                by2 = plsc.load_gather(vbf, [ci4 + 3])
                lbv = plsc.load_gather(vlb, [cis])
                plsc.store_scatter(rb, [tv * 4], bx1, mask=m0)
                plsc.store_scatter(rb, [tv * 4 + 1], by1, mask=m0)
                plsc.store_scatter(rb, [tv * 4 + 2], bx2, mask=m0)
                plsc.store_scatter(rb, [tv * 4 + 3], by2, mask=m0)
                plsc.store_scatter(rs, [tv], cs, mask=m0)
                plsc.store_scatter(rl, [tv], lbv, mask=m0)
                adv = valid.astype(jnp.int32)
                hp = hp + jnp.where(takeh, adv, 0)
                op = op + jnp.where(takeh, 0, adv)
                return (hp, op)

            lax.fori_loop(0, 2 * KCAP, mg_body, (jnp.int32(0), jnp.int32(0)))

            pltpu.sync_copy(rb, obh)
            pltpu.sync_copy(rs, osh)
            pltpu.sync_copy(rl, olh)


_mesh = plsc.VectorSubcoreMesh(core_axis_name="c", subcore_axis_name="s",
                               num_cores=2, num_subcores=16)

_OUT_TYPE = [
    jax.ShapeDtypeStruct((128,), jnp.float32),
    jax.ShapeDtypeStruct((32,), jnp.float32),
    jax.ShapeDtypeStruct((32,), jnp.int32),
]

_SCRATCH_TYPES = [
    pltpu.VMEM((4 * N,), jnp.float32),
    pltpu.VMEM((NPAD + LANES,), jnp.float32),
    pltpu.VMEM((NPAD + LANES,), jnp.int32),
    pltpu.VMEM((NPAD + LANES,), jnp.int32),
    pltpu.VMEM((NPAD + LANES,), jnp.int32),
    pltpu.VMEM((NPAD + LANES,), jnp.int32),
    pltpu.VMEM((NPAD + LANES,), jnp.float32),
    pltpu.VMEM((CPT * LANES,), jnp.float32),
    pltpu.VMEM((CPT * LANES,), jnp.int32),
    pltpu.VMEM((NCLS * LANES,), jnp.float32),
    pltpu.VMEM((NCLS * LANES,), jnp.int32),
    pltpu.VMEM((NCLS,), jnp.float32),
    pltpu.VMEM((NCLS,), jnp.int32),
    pltpu.VMEM((NCLS,), jnp.int32),
    pltpu.VMEM((128,), jnp.float32),
    pltpu.VMEM((32,), jnp.float32),
    pltpu.VMEM((32,), jnp.int32),
    pltpu.SemaphoreType.DMA,
    pltpu.SemaphoreType.DMA,
    pltpu.SemaphoreType.DMA,
    pltpu.VMEM_SHARED((NCLS * LANES,), jnp.float32),
    pltpu.VMEM_SHARED((NCLS * LANES,), jnp.int32),
]

_sc_call = pl.kernel(
    _nms_body,
    out_type=_OUT_TYPE,
    mesh=_mesh,
    compiler_params=pltpu.CompilerParams(needs_layout_passes=False),
    scratch_types=_SCRATCH_TYPES,
)


@jax.jit
def kernel(boxes, scores, labels):
    obf, osf, olf = _sc_call(boxes.reshape(-1), scores, labels)
    return obf[:120].reshape(30, 4), osf[:30], olf[:30]


# speedup vs baseline: 1.0176x; 1.0176x over previous
import jax
import jax.numpy as jnp
from jax import lax
from jax.experimental import pallas as pl
from jax.experimental.pallas import tpu as pltpu
from jax.experimental.pallas import tpu_sc as plsc

N = 5000
LANES = 16
NPAD = 5120
NCH = NPAD // LANES
NCLS = 80
HUMAN_IDX = 1
NMS_THRESH = 0.5
SCORE_THRESH = 0.2
KCAP = 15
TILES = 16
CPT = NCLS // TILES
NEGS = -3.0e38
DUMMY = 3.0e9
BIGI = 2**30


def _nms_body(bfh, sch, lbh, obh, osh, olh,
              vbf, vsc, vlb, rmidx, rmlab, midx, ms,
              kvs, kvi, gsc, gidx, heads_s, heads_i, ptrv, rb, rs, rl, lmax,
              sem1, sem2, sem3, ssc, sidx, smax):
    core = lax.axis_index("c")
    sub = lax.axis_index("s")
    lanes = lax.iota(jnp.int32, LANES)
    ones = lanes >= 0
    negs16 = jnp.full((LANES,), NEGS, jnp.float32)
    bigi16 = jnp.full((LANES,), BIGI, jnp.int32)

    @pl.when(core == 0)
    def _():
        cp1 = pltpu.async_copy(bfh, vbf, sem1)
        cp2 = pltpu.async_copy(sch, vsc.at[pl.ds(0, N)], sem2)
        cp3 = pltpu.async_copy(lbh, vlb.at[pl.ds(0, N)], sem3)
        with jax.named_scope("ph_dma"):
            cp2.wait()
            cp3.wait()
        negone = jnp.full((LANES,), -1.0, jnp.float32)
        negonei = jnp.full((LANES,), -1, jnp.int32)
        for k in range(8):
            plsc.store_compressed(vsc.at[pl.ds(N + 16 * k, LANES)], negone,
                                  mask=ones)
            plsc.store_compressed(vlb.at[pl.ds(N + 16 * k, LANES)], negonei,
                                  mask=ones)
        vsc[pl.ds(NPAD, LANES)] = negone
        vlb[pl.ds(NPAD, LANES)] = negonei

        lo = sub * CPT

        def rchunk(j, cntv):
            lab16 = vlb[pl.ds(j * LANES, LANES)]
            sc16 = vsc[pl.ds(j * LANES, LANES)]
            m = (lab16 >= lo) & (lab16 < lo + CPT) & (sc16 >= SCORE_THRESH)
            idx16 = j * LANES + lanes
            posi = cntv + plsc.cumsum(m.astype(jnp.int32)) - 1
            plsc.store_scatter(rmidx, [posi], idx16, mask=m)
            plsc.store_scatter(rmlab, [posi], lab16, mask=m)
            return cntv + plsc.all_reduce_population_count(m)

        def rscan(j, cntv):
            cntv = rchunk(2 * j, cntv)
            return rchunk(2 * j + 1, cntv)

        with jax.named_scope("ph_rscan"):
            cntv = lax.fori_loop(0, NCH // 2, rscan,
                                 jnp.zeros((LANES,), jnp.int32))
        rcnt = cntv[0]
        plsc.store_compressed(rmlab.at[pl.ds(rcnt, LANES)],
                              jnp.full((LANES,), -1, jnp.int32), mask=ones)
        rch = (rcnt + (LANES - 1)) >> 4

        MXCH = (4 * N) // LANES
        MPT = MXCH // TILES

        def mx_body(j, acc):
            base = (sub * MPT + j) * LANES
            return jnp.maximum(acc, vbf[pl.ds(base, LANES)])

        with jax.named_scope("ph_maxc"):
            cp1.wait()
            acc = lax.fori_loop(0, MPT, mx_body, negs16)

            @pl.when(sub == 0)
            def _():
                a2 = jnp.maximum(vbf[pl.ds(MPT * TILES * LANES, LANES)],
                                 vbf[pl.ds(MPT * TILES * LANES + LANES,
                                           LANES)])
                kvs[pl.ds(0, LANES)] = jnp.maximum(acc, a2)

            @pl.when(sub != 0)
            def _():
                kvs[pl.ds(0, LANES)] = acc

            pltpu.sync_copy(kvs.at[pl.ds(0, LANES)],
                            smax.at[pl.ds(sub * LANES, LANES)])
            plsc.subcore_barrier()
            pltpu.sync_copy(smax, lmax)
            macc = negs16
            for j in range(TILES):
                macc = jnp.maximum(macc, lmax[pl.ds(j * LANES, LANES)])
        maxc = jnp.max(macc) + jnp.float32(1.0)

        for k in range(CPT):
            c = lo + k
            off = c.astype(jnp.float32) * maxc

            def scan_body(j, cnt):
                lab16 = rmlab[pl.ds(j * LANES, LANES)]
                m = lab16 == c
                plsc.store_compressed(midx.at[pl.ds(cnt, LANES)],
                                      rmidx[pl.ds(j * LANES, LANES)], mask=m)
                return cnt + plsc.all_reduce_population_count(m)[0]

            with jax.named_scope("ph_l2scan"):
                cnt = lax.fori_loop(0, rch, scan_body, jnp.int32(0))
                nchk0 = (cnt + (LANES - 1)) >> 4

                def ms_body(j, _):
                    mi = midx[pl.ds(j * LANES, LANES)]
                    ms[pl.ds(j * LANES, LANES)] = plsc.load_gather(vsc, [mi])
                    return 0

                plsc.store_compressed(midx.at[pl.ds(cnt, LANES)],
                                      jnp.zeros((LANES,), jnp.int32),
                                      mask=ones)
                lax.fori_loop(0, nchk0, ms_body, 0)
            plsc.store_compressed(ms.at[pl.ds(cnt, LANES)], negs16, mask=ones)

            def cond(st):
                return (st[0] < cnt) & (st[1] < KCAP)

            def body(st):
                nproc, kcnt, kx1, ky1, kx2, ky2, kid, ksc = st
                nchk = (cnt + (LANES - 1)) >> 4

                def am_body(j, s):
                    bv, bp = s
                    v = ms[pl.ds(j * LANES, LANES)]
                    upd = v > bv
                    return jnp.where(upd, v, bv), jnp.where(upd, j, bp)

                bv, bp = lax.fori_loop(0, nchk, am_body,
                                       (negs16, jnp.zeros((LANES,), jnp.int32)))
                gmax = jnp.max(bv)
                posl = jnp.where(bv == gmax, bp * LANES + lanes, BIGI)
                pos = jnp.min(posl)
                posv = jnp.full((LANES,), pos, jnp.int32)
                plsc.store_scatter(ms, [posv], negs16, mask=lanes == 0)
                giv = plsc.load_gather(midx, [posv])
                g4 = giv * 4
                cx1 = plsc.load_gather(vbf, [g4]) + off
                cy1 = plsc.load_gather(vbf, [g4 + 1]) + off
                cx2 = plsc.load_gather(vbf, [g4 + 2]) + off
                cy2 = plsc.load_gather(vbf, [g4 + 3]) + off
                w = jnp.maximum(jnp.minimum(kx2, cx2) - jnp.maximum(kx1, cx1), 0.0)
                h = jnp.maximum(jnp.minimum(ky2, cy2) - jnp.maximum(ky1, cy1), 0.0)
                inter = w * h
                ka = (kx2 - kx1) * (ky2 - ky1)
                ca = (cx2 - cx1) * (cy2 - cy1)
                iou = inter / jnp.maximum(ka + ca - inter, jnp.float32(1e-9))
                sup = plsc.all_reduce_population_count(iou > NMS_THRESH)[0] > 0
                addm = jnp.logical_and(jnp.logical_not(sup), lanes == kcnt)
                kx1 = jnp.where(addm, cx1, kx1)
                ky1 = jnp.where(addm, cy1, ky1)
                kx2 = jnp.where(addm, cx2, kx2)
                ky2 = jnp.where(addm, cy2, ky2)
                kid = jnp.where(addm, giv, kid)
                ksc = jnp.where(addm, gmax, ksc)
                kcnt = kcnt + jnp.where(sup, 0, 1).astype(jnp.int32)
                return (nproc + 1, kcnt, kx1, ky1, kx2, ky2, kid, ksc)

            dummy16 = jnp.full((LANES,), DUMMY, jnp.float32)
            with jax.named_scope("ph_nms"):
                st = lax.while_loop(cond, body,
                                    (jnp.int32(0), jnp.int32(0),
                                     dummy16, dummy16, dummy16, dummy16,
                                     bigi16, negs16))
            kvs[pl.ds(k * LANES, LANES)] = st[7]
            kvi[pl.ds(k * LANES, LANES)] = st[6]

        pltpu.sync_copy(kvs, ssc.at[pl.ds(lo * LANES, CPT * LANES)])
        pltpu.sync_copy(kvi, sidx.at[pl.ds(lo * LANES, CPT * LANES)])

        plsc.subcore_barrier()

        @pl.when(sub == 0)
        def _():
          with jax.named_scope("ph_merge"):
            pltpu.sync_copy(ssc, gsc)
            pltpu.sync_copy(sidx, gidx)
            hs = gsc[pl.ds(HUMAN_IDX * LANES, LANES)]
            hi = gidx[pl.ds(HUMAN_IDX * LANES, LANES)]
            gsc[pl.ds(HUMAN_IDX * LANES, LANES)] = negs16
            for j in range(NCLS // LANES):
                rowv = (j * LANES + lanes) * LANES
                heads_s[pl.ds(j * LANES, LANES)] = plsc.load_gather(gsc, [rowv])
                heads_i[pl.ds(j * LANES, LANES)] = plsc.load_gather(gidx, [rowv])
            one16 = jnp.full((LANES,), 1, jnp.int32)
            for j in range(NCLS // LANES):
                ptrv[pl.ds(j * LANES, LANES)] = one16

            def ext_body(t, s):
                osc, oidx = s

                def hb(j, hst):
                    bv, bi, bp = hst
                    v = heads_s[pl.ds(j * LANES, LANES)]
                    iv = heads_i[pl.ds(j * LANES, LANES)]
                    upd = (v > bv) | ((v == bv) & (iv < bi))
                    return (jnp.where(upd, v, bv), jnp.where(upd, iv, bi),
                            jnp.where(upd, j, bp))

                bv, bi, bp = lax.fori_loop(0, NCLS // LANES, hb,
                                           (negs16, bigi16,
                                            jnp.zeros((LANES,), jnp.int32)))
                gmax = jnp.max(bv)
                gidw = jnp.min(jnp.where(bv == gmax, bi, BIGI))
                cls = jnp.min(jnp.where((bv == gmax) & (bi == gidw),
                                        bp * LANES + lanes, BIGI))
                clsv = jnp.full((LANES,), cls, jnp.int32)
                p = plsc.load_gather(ptrv, [clsv])
                plsc.store_scatter(ptrv, [clsv], p + 1, mask=lanes == 0)
                psafe = jnp.minimum(p, LANES - 1)
                hv = plsc.load_gather(gsc, [clsv * LANES + psafe])
                hiv = plsc.load_gather(gidx, [clsv * LANES + psafe])
                hv = jnp.where(p >= LANES, negs16, hv)
                plsc.store_scatter(heads_s, [clsv], hv, mask=lanes == 0)
                plsc.store_scatter(heads_i, [clsv], hiv, mask=lanes == 0)
                valid = gmax > jnp.float32(-1.0e37)
                osc = jnp.where((lanes == t) & valid, gmax, osc)
                oidx = jnp.where((lanes == t) & valid, gidw, oidx)
                return (osc, oidx)

            osc, oidx = lax.fori_loop(0, KCAP, ext_body, (negs16, bigi16))

            heads_s[pl.ds(0, LANES)] = hs
            heads_i[pl.ds(0, LANES)] = hi
            heads_s[pl.ds(LANES, LANES)] = osc
            heads_i[pl.ds(LANES, LANES)] = oidx

            zf16 = jnp.zeros((LANES,), jnp.float32)
            for j in range(8):
                rb[pl.ds(j * LANES, LANES)] = zf16
            rs[pl.ds(0, LANES)] = zf16
            rs[pl.ds(LANES, LANES)] = zf16
            neg1 = jnp.full((LANES,), -1, jnp.int32)
            rl[pl.ds(0, LANES)] = neg1
            rl[pl.ds(LANES, LANES)] = neg1

            def mg_body(t, s):
                hp, op = s
                hpv = jnp.full((LANES,), hp, jnp.int32)
                opv = jnp.full((LANES,), op + LANES, jnp.int32)
                hv = plsc.load_gather(heads_s, [hpv])
                hiv = plsc.load_gather(heads_i, [hpv])
                ov = plsc.load_gather(heads_s, [opv])
                oiv = plsc.load_gather(heads_i, [opv])
                hvs = hv[0]
                ovs = ov[0]
                his = hiv[0]
                ois = oiv[0]
                takeh = (hvs > ovs) | ((hvs == ovs) & (his < ois))
                cs = jnp.where(takeh, hv, ov)
                ci = jnp.where(takeh, hiv, oiv)
                valid = cs[0] > jnp.float32(-1.0e37)
                cis = jnp.where(valid, ci, jnp.zeros((LANES,), jnp.int32))
                ci4 = cis * 4
                m0 = (lanes == 0) & valid
                tv = jnp.full((LANES,), t, jnp.int32)
                bx1 = plsc.load_gather(vbf, [ci4])
                by1 = plsc.load_gather(vbf, [ci4 + 1])
                bx2 = plsc.load_gather(vbf, [ci4 + 2])
                by2 = plsc.load_gather(vbf, [ci4 + 3])
                lbv = plsc.load_gather(vlb, [cis])
                plsc.store_scatter(rb, [tv * 4], bx1, mask=m0)
                plsc.store_scatter(rb, [tv * 4 + 1], by1, mask=m0)
                plsc.store_scatter(rb, [tv * 4 + 2], bx2, mask=m0)
                plsc.store_scatter(rb, [tv * 4 + 3], by2, mask=m0)
                plsc.store_scatter(rs, [tv], cs, mask=m0)
                plsc.store_scatter(rl, [tv], lbv, mask=m0)
                adv = valid.astype(jnp.int32)
                hp = hp + jnp.where(takeh, adv, 0)
                op = op + jnp.where(takeh, 0, adv)
                return (hp, op)

            lax.fori_loop(0, 2 * KCAP, mg_body, (jnp.int32(0), jnp.int32(0)))

            pltpu.sync_copy(rb, obh)
            pltpu.sync_copy(rs, osh)
            pltpu.sync_copy(rl, olh)


_mesh = plsc.VectorSubcoreMesh(core_axis_name="c", subcore_axis_name="s",
                               num_cores=2, num_subcores=16)

_OUT_TYPE = [
    jax.ShapeDtypeStruct((128,), jnp.float32),
    jax.ShapeDtypeStruct((32,), jnp.float32),
    jax.ShapeDtypeStruct((32,), jnp.int32),
]

_SCRATCH_TYPES = [
    pltpu.VMEM((4 * N,), jnp.float32),
    pltpu.VMEM((NPAD + LANES,), jnp.float32),
    pltpu.VMEM((NPAD + LANES,), jnp.int32),
    pltpu.VMEM((NPAD + LANES,), jnp.int32),
    pltpu.VMEM((NPAD + LANES,), jnp.int32),
    pltpu.VMEM((NPAD + LANES,), jnp.int32),
    pltpu.VMEM((NPAD + LANES,), jnp.float32),
    pltpu.VMEM((CPT * LANES,), jnp.float32),
    pltpu.VMEM((CPT * LANES,), jnp.int32),
    pltpu.VMEM((NCLS * LANES,), jnp.float32),
    pltpu.VMEM((NCLS * LANES,), jnp.int32),
    pltpu.VMEM((NCLS,), jnp.float32),
    pltpu.VMEM((NCLS,), jnp.int32),
    pltpu.VMEM((NCLS,), jnp.int32),
    pltpu.VMEM((128,), jnp.float32),
    pltpu.VMEM((32,), jnp.float32),
    pltpu.VMEM((32,), jnp.int32),
    pltpu.VMEM((TILES * LANES,), jnp.float32),
    pltpu.SemaphoreType.DMA,
    pltpu.SemaphoreType.DMA,
    pltpu.SemaphoreType.DMA,
    pltpu.VMEM_SHARED((NCLS * LANES,), jnp.float32),
    pltpu.VMEM_SHARED((NCLS * LANES,), jnp.int32),
    pltpu.VMEM_SHARED((TILES * LANES,), jnp.float32),
]

_sc_call = pl.kernel(
    _nms_body,
    out_type=_OUT_TYPE,
    mesh=_mesh,
    compiler_params=pltpu.CompilerParams(needs_layout_passes=False),
    scratch_types=_SCRATCH_TYPES,
)


@jax.jit
def kernel(boxes, scores, labels):
    obf, osf, olf = _sc_call(boxes.reshape(-1), scores, labels)
    return obf[:120].reshape(30, 4), osf[:30], olf[:30]


# speedup vs baseline: 1.0417x; 1.0237x over previous
import jax
import jax.numpy as jnp
from jax import lax
from jax.experimental import pallas as pl
from jax.experimental.pallas import tpu as pltpu
from jax.experimental.pallas import tpu_sc as plsc

N = 5000
LANES = 16
NPAD = 5120
NCH = NPAD // LANES
NCLS = 80
HUMAN_IDX = 1
NMS_THRESH = 0.5
SCORE_THRESH = 0.2
KCAP = 15
TILES = 16
CPT = NCLS // TILES
NEGS = -3.0e38
DUMMY = 3.0e9
BIGI = 2**30


def _nms_body(bfh, sch, lbh, obh, osh, olh,
              vbf, vsc, vlb, rmidx, rmlab, midx, ms,
              kvs, kvi, gsc, gidx, heads_s, heads_i, ptrv, rb, rs, rl, lmax,
              sem1, sem2, sem3, ssc, sidx, smax):
    core = lax.axis_index("c")
    sub = lax.axis_index("s")
    lanes = lax.iota(jnp.int32, LANES)
    ones = lanes >= 0
    negs16 = jnp.full((LANES,), NEGS, jnp.float32)
    bigi16 = jnp.full((LANES,), BIGI, jnp.int32)

    @pl.when(core == 0)
    def _():
        cp1 = pltpu.async_copy(bfh, vbf, sem1)
        cp2 = pltpu.async_copy(sch, vsc.at[pl.ds(0, N)], sem2)
        cp3 = pltpu.async_copy(lbh, vlb.at[pl.ds(0, N)], sem3)
        with jax.named_scope("ph_dma"):
            cp2.wait()
            cp3.wait()
        negone = jnp.full((LANES,), -1.0, jnp.float32)
        negonei = jnp.full((LANES,), -1, jnp.int32)
        for k in range(8):
            plsc.store_compressed(vsc.at[pl.ds(N + 16 * k, LANES)], negone,
                                  mask=ones)
            plsc.store_compressed(vlb.at[pl.ds(N + 16 * k, LANES)], negonei,
                                  mask=ones)
        vsc[pl.ds(NPAD, LANES)] = negone
        vlb[pl.ds(NPAD, LANES)] = negonei

        lo = sub * CPT

        def rchunk(j, cnt):
            lab16 = vlb[pl.ds(j * LANES, LANES)]
            sc16 = vsc[pl.ds(j * LANES, LANES)]
            m = (lab16 >= lo) & (lab16 < lo + CPT) & (sc16 >= SCORE_THRESH)
            idx16 = j * LANES + lanes
            plsc.store_compressed(rmidx.at[pl.ds(cnt, LANES)], idx16, mask=m)
            plsc.store_compressed(rmlab.at[pl.ds(cnt, LANES)], lab16, mask=m)
            return cnt + plsc.all_reduce_population_count(m)[0]

        def rscan(j, cnt):
            cnt = rchunk(2 * j, cnt)
            return rchunk(2 * j + 1, cnt)

        with jax.named_scope("ph_rscan"):
            rcnt = lax.fori_loop(0, NCH // 2, rscan, jnp.int32(0))
        plsc.store_compressed(rmlab.at[pl.ds(rcnt, LANES)],
                              jnp.full((LANES,), -1, jnp.int32), mask=ones)
        rch = (rcnt + (LANES - 1)) >> 4

        MXCH = (4 * N) // LANES
        MPT = MXCH // TILES

        def mx_body(j, acc):
            base = (sub * MPT + j) * LANES
            return jnp.maximum(acc, vbf[pl.ds(base, LANES)])

        with jax.named_scope("ph_maxc"):
            cp1.wait()
            acc = lax.fori_loop(0, MPT, mx_body, negs16)

            @pl.when(sub == 0)
            def _():
                a2 = jnp.maximum(vbf[pl.ds(MPT * TILES * LANES, LANES)],
                                 vbf[pl.ds(MPT * TILES * LANES + LANES,
                                           LANES)])
                kvs[pl.ds(0, LANES)] = jnp.maximum(acc, a2)

            @pl.when(sub != 0)
            def _():
                kvs[pl.ds(0, LANES)] = acc

            pltpu.sync_copy(kvs.at[pl.ds(0, LANES)],
                            smax.at[pl.ds(sub * LANES, LANES)])
            plsc.subcore_barrier()
            pltpu.sync_copy(smax, lmax)
            macc = negs16
            for j in range(TILES):
                macc = jnp.maximum(macc, lmax[pl.ds(j * LANES, LANES)])
        maxc = jnp.max(macc) + jnp.float32(1.0)

        for k in range(CPT):
            c = lo + k
            off = c.astype(jnp.float32) * maxc

            def scan_body(j, cnt):
                lab16 = rmlab[pl.ds(j * LANES, LANES)]
                m = lab16 == c
                plsc.store_compressed(midx.at[pl.ds(cnt, LANES)],
                                      rmidx[pl.ds(j * LANES, LANES)], mask=m)
                return cnt + plsc.all_reduce_population_count(m)[0]

            with jax.named_scope("ph_l2scan"):
                cnt = lax.fori_loop(0, rch, scan_body, jnp.int32(0))
                nchk0 = (cnt + (LANES - 1)) >> 4

                def ms_body(j, _):
                    mi = midx[pl.ds(j * LANES, LANES)]
                    ms[pl.ds(j * LANES, LANES)] = plsc.load_gather(vsc, [mi])
                    return 0

                plsc.store_compressed(midx.at[pl.ds(cnt, LANES)],
                                      jnp.zeros((LANES,), jnp.int32),
                                      mask=ones)
                lax.fori_loop(0, nchk0, ms_body, 0)
            plsc.store_compressed(ms.at[pl.ds(cnt, LANES)], negs16, mask=ones)

            def cond(st):
                return (st[0] < cnt) & (st[1] < KCAP)

            def body(st):
                nproc, kcnt, kx1, ky1, kx2, ky2, kid, ksc = st
                nchk = (cnt + (LANES - 1)) >> 4

                def am_body(j, s):
                    bv, bp = s
                    v = ms[pl.ds(j * LANES, LANES)]
                    upd = v > bv
                    return jnp.where(upd, v, bv), jnp.where(upd, j, bp)

                bv, bp = lax.fori_loop(0, nchk, am_body,
                                       (negs16, jnp.zeros((LANES,), jnp.int32)))
                gmax = jnp.max(bv)
                posl = jnp.where(bv == gmax, bp * LANES + lanes, BIGI)
                pos = jnp.min(posl)
                posv = jnp.full((LANES,), pos, jnp.int32)
                plsc.store_scatter(ms, [posv], negs16, mask=lanes == 0)
                giv = plsc.load_gather(midx, [posv])
                g4 = giv * 4
                cx1 = plsc.load_gather(vbf, [g4]) + off
                cy1 = plsc.load_gather(vbf, [g4 + 1]) + off
                cx2 = plsc.load_gather(vbf, [g4 + 2]) + off
                cy2 = plsc.load_gather(vbf, [g4 + 3]) + off
                w = jnp.maximum(jnp.minimum(kx2, cx2) - jnp.maximum(kx1, cx1), 0.0)
                h = jnp.maximum(jnp.minimum(ky2, cy2) - jnp.maximum(ky1, cy1), 0.0)
                inter = w * h
                ka = (kx2 - kx1) * (ky2 - ky1)
                ca = (cx2 - cx1) * (cy2 - cy1)
                iou = inter / jnp.maximum(ka + ca - inter, jnp.float32(1e-9))
                sup = plsc.all_reduce_population_count(iou > NMS_THRESH)[0] > 0
                addm = jnp.logical_and(jnp.logical_not(sup), lanes == kcnt)
                kx1 = jnp.where(addm, cx1, kx1)
                ky1 = jnp.where(addm, cy1, ky1)
                kx2 = jnp.where(addm, cx2, kx2)
                ky2 = jnp.where(addm, cy2, ky2)
                kid = jnp.where(addm, giv, kid)
                ksc = jnp.where(addm, gmax, ksc)
                kcnt = kcnt + jnp.where(sup, 0, 1).astype(jnp.int32)
                return (nproc + 1, kcnt, kx1, ky1, kx2, ky2, kid, ksc)

            dummy16 = jnp.full((LANES,), DUMMY, jnp.float32)
            with jax.named_scope("ph_nms"):
                st = lax.while_loop(cond, body,
                                    (jnp.int32(0), jnp.int32(0),
                                     dummy16, dummy16, dummy16, dummy16,
                                     bigi16, negs16))
            kvs[pl.ds(k * LANES, LANES)] = st[7]
            kvi[pl.ds(k * LANES, LANES)] = st[6]

        pltpu.sync_copy(kvs, ssc.at[pl.ds(lo * LANES, CPT * LANES)])
        pltpu.sync_copy(kvi, sidx.at[pl.ds(lo * LANES, CPT * LANES)])

        plsc.subcore_barrier()

        @pl.when(sub == 0)
        def _():
          with jax.named_scope("ph_merge"):
            pltpu.sync_copy(ssc, gsc)
            pltpu.sync_copy(sidx, gidx)
            hs = gsc[pl.ds(HUMAN_IDX * LANES, LANES)]
            hi = gidx[pl.ds(HUMAN_IDX * LANES, LANES)]
            gsc[pl.ds(HUMAN_IDX * LANES, LANES)] = negs16
            for j in range(NCLS // LANES):
                rowv = (j * LANES + lanes) * LANES
                heads_s[pl.ds(j * LANES, LANES)] = plsc.load_gather(gsc, [rowv])
                heads_i[pl.ds(j * LANES, LANES)] = plsc.load_gather(gidx, [rowv])
            one16 = jnp.full((LANES,), 1, jnp.int32)
            for j in range(NCLS // LANES):
                ptrv[pl.ds(j * LANES, LANES)] = one16

            def ext_body(t, s):
                osc, oidx = s

                def hb(j, hst):
                    bv, bi, bp = hst
                    v = heads_s[pl.ds(j * LANES, LANES)]
                    iv = heads_i[pl.ds(j * LANES, LANES)]
                    upd = (v > bv) | ((v == bv) & (iv < bi))
                    return (jnp.where(upd, v, bv), jnp.where(upd, iv, bi),
                            jnp.where(upd, j, bp))

                bv, bi, bp = lax.fori_loop(0, NCLS // LANES, hb,
                                           (negs16, bigi16,
                                            jnp.zeros((LANES,), jnp.int32)))
                gmax = jnp.max(bv)
                gidw = jnp.min(jnp.where(bv == gmax, bi, BIGI))
                cls = jnp.min(jnp.where((bv == gmax) & (bi == gidw),
                                        bp * LANES + lanes, BIGI))
                clsv = jnp.full((LANES,), cls, jnp.int32)
                p = plsc.load_gather(ptrv, [clsv])
                plsc.store_scatter(ptrv, [clsv], p + 1, mask=lanes == 0)
                psafe = jnp.minimum(p, LANES - 1)
                hv = plsc.load_gather(gsc, [clsv * LANES + psafe])
                hiv = plsc.load_gather(gidx, [clsv * LANES + psafe])
                hv = jnp.where(p >= LANES, negs16, hv)
                plsc.store_scatter(heads_s, [clsv], hv, mask=lanes == 0)
                plsc.store_scatter(heads_i, [clsv], hiv, mask=lanes == 0)
                valid = gmax > jnp.float32(-1.0e37)
                osc = jnp.where((lanes == t) & valid, gmax, osc)
                oidx = jnp.where((lanes == t) & valid, gidw, oidx)
                return (osc, oidx)

            osc, oidx = lax.fori_loop(0, KCAP, ext_body, (negs16, bigi16))

            heads_s[pl.ds(0, LANES)] = hs
            heads_i[pl.ds(0, LANES)] = hi
            heads_s[pl.ds(LANES, LANES)] = osc
            heads_i[pl.ds(LANES, LANES)] = oidx

            zf16 = jnp.zeros((LANES,), jnp.float32)
            for j in range(8):
                rb[pl.ds(j * LANES, LANES)] = zf16
            rs[pl.ds(0, LANES)] = zf16
            rs[pl.ds(LANES, LANES)] = zf16
            neg1 = jnp.full((LANES,), -1, jnp.int32)
            rl[pl.ds(0, LANES)] = neg1
            rl[pl.ds(LANES, LANES)] = neg1

            def mg_body(t, s):
                hp, op = s
                hpv = jnp.full((LANES,), hp, jnp.int32)
                opv = jnp.full((LANES,), op + LANES, jnp.int32)
                hv = plsc.load_gather(heads_s, [hpv])
                hiv = plsc.load_gather(heads_i, [hpv])
                ov = plsc.load_gather(heads_s, [opv])
                oiv = plsc.load_gather(heads_i, [opv])
                hvs = hv[0]
                ovs = ov[0]
                his = hiv[0]
                ois = oiv[0]
                takeh = (hvs > ovs) | ((hvs == ovs) & (his < ois))
                cs = jnp.where(takeh, hv, ov)
                ci = jnp.where(takeh, hiv, oiv)
                valid = cs[0] > jnp.float32(-1.0e37)
                cis = jnp.where(valid, ci, jnp.zeros((LANES,), jnp.int32))
                ci4 = cis * 4
                m0 = (lanes == 0) & valid
                tv = jnp.full((LANES,), t, jnp.int32)
                bx1 = plsc.load_gather(vbf, [ci4])
                by1 = plsc.load_gather(vbf, [ci4 + 1])
                bx2 = plsc.load_gather(vbf, [ci4 + 2])
                by2 = plsc.load_gather(vbf, [ci4 + 3])
                lbv = plsc.load_gather(vlb, [cis])
                plsc.store_scatter(rb, [tv * 4], bx1, mask=m0)
                plsc.store_scatter(rb, [tv * 4 + 1], by1, mask=m0)
                plsc.store_scatter(rb, [tv * 4 + 2], bx2, mask=m0)
                plsc.store_scatter(rb, [tv * 4 + 3], by2, mask=m0)
                plsc.store_scatter(rs, [tv], cs, mask=m0)
                plsc.store_scatter(rl, [tv], lbv, mask=m0)
                adv = valid.astype(jnp.int32)
                hp = hp + jnp.where(takeh, adv, 0)
                op = op + jnp.where(takeh, 0, adv)
                return (hp, op)

            lax.fori_loop(0, 2 * KCAP, mg_body, (jnp.int32(0), jnp.int32(0)))

            pltpu.sync_copy(rb, obh)
            pltpu.sync_copy(rs, osh)
            pltpu.sync_copy(rl, olh)


_mesh = plsc.VectorSubcoreMesh(core_axis_name="c", subcore_axis_name="s",
                               num_cores=2, num_subcores=16)

_OUT_TYPE = [
    jax.ShapeDtypeStruct((128,), jnp.float32),
    jax.ShapeDtypeStruct((32,), jnp.float32),
    jax.ShapeDtypeStruct((32,), jnp.int32),
]

_SCRATCH_TYPES = [
    pltpu.VMEM((4 * N,), jnp.float32),
    pltpu.VMEM((NPAD + LANES,), jnp.float32),
    pltpu.VMEM((NPAD + LANES,), jnp.int32),
    pltpu.VMEM((NPAD + LANES,), jnp.int32),
    pltpu.VMEM((NPAD + LANES,), jnp.int32),
    pltpu.VMEM((NPAD + LANES,), jnp.int32),
    pltpu.VMEM((NPAD + LANES,), jnp.float32),
    pltpu.VMEM((CPT * LANES,), jnp.float32),
    pltpu.VMEM((CPT * LANES,), jnp.int32),
    pltpu.VMEM((NCLS * LANES,), jnp.float32),
    pltpu.VMEM((NCLS * LANES,), jnp.int32),
    pltpu.VMEM((NCLS,), jnp.float32),
    pltpu.VMEM((NCLS,), jnp.int32),
    pltpu.VMEM((NCLS,), jnp.int32),
    pltpu.VMEM((128,), jnp.float32),
    pltpu.VMEM((32,), jnp.float32),
    pltpu.VMEM((32,), jnp.int32),
    pltpu.VMEM((TILES * LANES,), jnp.float32),
    pltpu.SemaphoreType.DMA,
    pltpu.SemaphoreType.DMA,
    pltpu.SemaphoreType.DMA,
    pltpu.VMEM_SHARED((NCLS * LANES,), jnp.float32),
    pltpu.VMEM_SHARED((NCLS * LANES,), jnp.int32),
    pltpu.VMEM_SHARED((TILES * LANES,), jnp.float32),
]

_sc_call = pl.kernel(
    _nms_body,
    out_type=_OUT_TYPE,
    mesh=_mesh,
    compiler_params=pltpu.CompilerParams(needs_layout_passes=False),
    scratch_types=_SCRATCH_TYPES,
)


@jax.jit
def kernel(boxes, scores, labels):
    obf, osf, olf = _sc_call(boxes.reshape(-1), scores, labels)
    return obf[:120].reshape(30, 4), osf[:30], olf[:30]


# speedup vs baseline: 1.0670x; 1.0243x over previous
import jax
import jax.numpy as jnp
from jax import lax
from jax.experimental import pallas as pl
from jax.experimental.pallas import tpu as pltpu
from jax.experimental.pallas import tpu_sc as plsc

N = 5000
LANES = 16
NPAD = 5120
NCH = NPAD // LANES
NCLS = 80
HUMAN_IDX = 1
NMS_THRESH = 0.5
SCORE_THRESH = 0.2
KCAP = 15
TILES = 16
CPT = NCLS // TILES
NEGS = -3.0e38
DUMMY = 3.0e9
BIGI = 2**30


def _nms_body(bfh, sch, lbh, obh, osh, olh,
              vbf, vsc, vlb, rmidx, rmlab, midx, ms,
              kvs, kvi, gsc, gidx, heads_s, heads_i, ptrv, rb, rs, rl, lmax,
              sem1, sem2, sem3, ssc, sidx, smax):
    core = lax.axis_index("c")
    sub = lax.axis_index("s")
    lanes = lax.iota(jnp.int32, LANES)
    ones = lanes >= 0
    negs16 = jnp.full((LANES,), NEGS, jnp.float32)
    bigi16 = jnp.full((LANES,), BIGI, jnp.int32)

    @pl.when(core == 0)
    def _():
        cp1 = pltpu.async_copy(bfh, vbf, sem1)
        cp2 = pltpu.async_copy(sch, vsc.at[pl.ds(0, N)], sem2)
        cp3 = pltpu.async_copy(lbh, vlb.at[pl.ds(0, N)], sem3)
        cp2.wait()
        cp3.wait()
        negone = jnp.full((LANES,), -1.0, jnp.float32)
        negonei = jnp.full((LANES,), -1, jnp.int32)
        for k in range(8):
            plsc.store_compressed(vsc.at[pl.ds(N + 16 * k, LANES)], negone,
                                  mask=ones)
            plsc.store_compressed(vlb.at[pl.ds(N + 16 * k, LANES)], negonei,
                                  mask=ones)
        vsc[pl.ds(NPAD, LANES)] = negone
        vlb[pl.ds(NPAD, LANES)] = negonei

        lo = sub * CPT

        def rchunk(j, cnt):
            lab16 = vlb[pl.ds(j * LANES, LANES)]
            sc16 = vsc[pl.ds(j * LANES, LANES)]
            m = (lab16 >= lo) & (lab16 < lo + CPT) & (sc16 >= SCORE_THRESH)
            idx16 = j * LANES + lanes
            plsc.store_compressed(rmidx.at[pl.ds(cnt, LANES)], idx16, mask=m)
            plsc.store_compressed(rmlab.at[pl.ds(cnt, LANES)], lab16, mask=m)
            return cnt + plsc.all_reduce_population_count(m)[0]

        def rscan(j, cnt):
            cnt = rchunk(2 * j, cnt)
            return rchunk(2 * j + 1, cnt)

        with jax.named_scope("ph_rscan"):
            rcnt = lax.fori_loop(0, NCH // 2, rscan, jnp.int32(0))
        plsc.store_compressed(rmlab.at[pl.ds(rcnt, LANES)],
                              jnp.full((LANES,), -1, jnp.int32), mask=ones)
        rch = (rcnt + (LANES - 1)) >> 4

        MXCH = (4 * N) // LANES
        MPT = MXCH // TILES

        def mx_body(j, acc):
            base = (sub * MPT + j) * LANES
            return jnp.maximum(acc, vbf[pl.ds(base, LANES)])

        with jax.named_scope("ph_maxc"):
            cp1.wait()
            acc = lax.fori_loop(0, MPT, mx_body, negs16)

            @pl.when(sub == 0)
            def _():
                a2 = jnp.maximum(vbf[pl.ds(MPT * TILES * LANES, LANES)],
                                 vbf[pl.ds(MPT * TILES * LANES + LANES,
                                           LANES)])
                kvs[pl.ds(0, LANES)] = jnp.maximum(acc, a2)

            @pl.when(sub != 0)
            def _():
                kvs[pl.ds(0, LANES)] = acc

            pltpu.sync_copy(kvs.at[pl.ds(0, LANES)],
                            smax.at[pl.ds(sub * LANES, LANES)])
            plsc.subcore_barrier()
            pltpu.sync_copy(smax, lmax)
            macc = negs16
            for j in range(TILES):
                macc = jnp.maximum(macc, lmax[pl.ds(j * LANES, LANES)])
        maxc = jnp.max(macc) + jnp.float32(1.0)

        for k in range(CPT):
            c = lo + k
            off = c.astype(jnp.float32) * maxc

            def scan_body(j, cnt):
                lab16 = rmlab[pl.ds(j * LANES, LANES)]
                m = lab16 == c
                plsc.store_compressed(midx.at[pl.ds(cnt, LANES)],
                                      rmidx[pl.ds(j * LANES, LANES)], mask=m)
                return cnt + plsc.all_reduce_population_count(m)[0]

            with jax.named_scope("ph_l2scan"):
                cnt = lax.fori_loop(0, rch, scan_body, jnp.int32(0))
                nchk0 = (cnt + (LANES - 1)) >> 4

                def ms_body(j, _):
                    mi = midx[pl.ds(j * LANES, LANES)]
                    ms[pl.ds(j * LANES, LANES)] = plsc.load_gather(vsc, [mi])
                    return 0

                plsc.store_compressed(midx.at[pl.ds(cnt, LANES)],
                                      jnp.zeros((LANES,), jnp.int32),
                                      mask=ones)
                lax.fori_loop(0, nchk0, ms_body, 0)
            plsc.store_compressed(ms.at[pl.ds(cnt, LANES)], negs16, mask=ones)
            plsc.store_compressed(ms.at[pl.ds(cnt + LANES, LANES)], negs16,
                                  mask=ones)

            def cond(st):
                return (st[0] < cnt) & (st[1] < KCAP)

            def body(st):
                nproc, kcnt, kx1, ky1, kx2, ky2, kid, ksc = st
                nchk2 = (cnt + (2 * LANES - 1)) >> 5

                def am_body(j, s):
                    bv, bp = s
                    v = ms[pl.ds(2 * j * LANES, LANES)]
                    upd = v > bv
                    bv = jnp.where(upd, v, bv)
                    bp = jnp.where(upd, 2 * j, bp)
                    v = ms[pl.ds((2 * j + 1) * LANES, LANES)]
                    upd = v > bv
                    return jnp.where(upd, v, bv), jnp.where(upd, 2 * j + 1, bp)

                bv, bp = lax.fori_loop(0, nchk2, am_body,
                                       (negs16, jnp.zeros((LANES,), jnp.int32)))
                gmax = jnp.max(bv)
                posl = jnp.where(bv == gmax, bp * LANES + lanes, BIGI)
                pos = jnp.min(posl)
                posv = jnp.full((LANES,), pos, jnp.int32)
                plsc.store_scatter(ms, [posv], negs16, mask=lanes == 0)
                giv = plsc.load_gather(midx, [posv])
                g4 = giv * 4
                cx1 = plsc.load_gather(vbf, [g4]) + off
                cy1 = plsc.load_gather(vbf, [g4 + 1]) + off
                cx2 = plsc.load_gather(vbf, [g4 + 2]) + off
                cy2 = plsc.load_gather(vbf, [g4 + 3]) + off
                w = jnp.maximum(jnp.minimum(kx2, cx2) - jnp.maximum(kx1, cx1), 0.0)
                h = jnp.maximum(jnp.minimum(ky2, cy2) - jnp.maximum(ky1, cy1), 0.0)
                inter = w * h
                ka = (kx2 - kx1) * (ky2 - ky1)
                ca = (cx2 - cx1) * (cy2 - cy1)
                iou = inter / jnp.maximum(ka + ca - inter, jnp.float32(1e-9))
                sup = plsc.all_reduce_population_count(iou > NMS_THRESH)[0] > 0
                addm = jnp.logical_and(jnp.logical_not(sup), lanes == kcnt)
                kx1 = jnp.where(addm, cx1, kx1)
                ky1 = jnp.where(addm, cy1, ky1)
                kx2 = jnp.where(addm, cx2, kx2)
                ky2 = jnp.where(addm, cy2, ky2)
                kid = jnp.where(addm, giv, kid)
                ksc = jnp.where(addm, gmax, ksc)
                kcnt = kcnt + jnp.where(sup, 0, 1).astype(jnp.int32)
                return (nproc + 1, kcnt, kx1, ky1, kx2, ky2, kid, ksc)

            dummy16 = jnp.full((LANES,), DUMMY, jnp.float32)
            with jax.named_scope("ph_nms"):
                st = lax.while_loop(cond, body,
                                    (jnp.int32(0), jnp.int32(0),
                                     dummy16, dummy16, dummy16, dummy16,
                                     bigi16, negs16))
            kvs[pl.ds(k * LANES, LANES)] = st[7]
            kvi[pl.ds(k * LANES, LANES)] = st[6]

        pltpu.sync_copy(kvs, ssc.at[pl.ds(lo * LANES, CPT * LANES)])
        pltpu.sync_copy(kvi, sidx.at[pl.ds(lo * LANES, CPT * LANES)])

        plsc.subcore_barrier()

        @pl.when(sub == 0)
        def _():
          with jax.named_scope("ph_merge"):
            pltpu.sync_copy(ssc, gsc)
            pltpu.sync_copy(sidx, gidx)
            hs = gsc[pl.ds(HUMAN_IDX * LANES, LANES)]
            hi = gidx[pl.ds(HUMAN_IDX * LANES, LANES)]
            gsc[pl.ds(HUMAN_IDX * LANES, LANES)] = negs16
            for j in range(NCLS // LANES):
                rowv = (j * LANES + lanes) * LANES
                heads_s[pl.ds(j * LANES, LANES)] = plsc.load_gather(gsc, [rowv])
                heads_i[pl.ds(j * LANES, LANES)] = plsc.load_gather(gidx, [rowv])
            one16 = jnp.full((LANES,), 1, jnp.int32)
            for j in range(NCLS // LANES):
                ptrv[pl.ds(j * LANES, LANES)] = one16

            def ext_body(t, s):
                osc, oidx = s

                bv, bi, bp = (negs16, bigi16, jnp.zeros((LANES,), jnp.int32))
                for j in range(NCLS // LANES):
                    v = heads_s[pl.ds(j * LANES, LANES)]
                    iv = heads_i[pl.ds(j * LANES, LANES)]
                    upd = (v > bv) | ((v == bv) & (iv < bi))
                    bv, bi, bp = (jnp.where(upd, v, bv),
                                  jnp.where(upd, iv, bi),
                                  jnp.where(upd, j, bp))
                gmax = jnp.max(bv)
                gidw = jnp.min(jnp.where(bv == gmax, bi, BIGI))
                cls = jnp.min(jnp.where((bv == gmax) & (bi == gidw),
                                        bp * LANES + lanes, BIGI))
                clsv = jnp.full((LANES,), cls, jnp.int32)
                p = plsc.load_gather(ptrv, [clsv])
                plsc.store_scatter(ptrv, [clsv], p + 1, mask=lanes == 0)
                psafe = jnp.minimum(p, LANES - 1)
                hv = plsc.load_gather(gsc, [clsv * LANES + psafe])
                hiv = plsc.load_gather(gidx, [clsv * LANES + psafe])
                hv = jnp.where(p >= LANES, negs16, hv)
                plsc.store_scatter(heads_s, [clsv], hv, mask=lanes == 0)
                plsc.store_scatter(heads_i, [clsv], hiv, mask=lanes == 0)
                valid = gmax > jnp.float32(-1.0e37)
                osc = jnp.where((lanes == t) & valid, gmax, osc)
                oidx = jnp.where((lanes == t) & valid, gidw, oidx)
                return (osc, oidx)

            osc, oidx = lax.fori_loop(0, KCAP, ext_body, (negs16, bigi16))

            heads_s[pl.ds(0, LANES)] = hs
            heads_i[pl.ds(0, LANES)] = hi
            heads_s[pl.ds(LANES, LANES)] = osc
            heads_i[pl.ds(LANES, LANES)] = oidx

            zf16 = jnp.zeros((LANES,), jnp.float32)
            for j in range(8):
                rb[pl.ds(j * LANES, LANES)] = zf16
            rs[pl.ds(0, LANES)] = zf16
            rs[pl.ds(LANES, LANES)] = zf16
            neg1 = jnp.full((LANES,), -1, jnp.int32)
            rl[pl.ds(0, LANES)] = neg1
            rl[pl.ds(LANES, LANES)] = neg1

            def mg_body(t, s):
                hp, op = s
                hpv = jnp.full((LANES,), hp, jnp.int32)
                opv = jnp.full((LANES,), op + LANES, jnp.int32)
                hv = plsc.load_gather(heads_s, [hpv])
                hiv = plsc.load_gather(heads_i, [hpv])
                ov = plsc.load_gather(heads_s, [opv])
                oiv = plsc.load_gather(heads_i, [opv])
                hvs = hv[0]
                ovs = ov[0]
                his = hiv[0]
                ois = oiv[0]
                takeh = (hvs > ovs) | ((hvs == ovs) & (his < ois))
                cs = jnp.where(takeh, hv, ov)
                ci = jnp.where(takeh, hiv, oiv)
                valid = cs[0] > jnp.float32(-1.0e37)
                cis = jnp.where(valid, ci, jnp.zeros((LANES,), jnp.int32))
                ci4 = cis * 4
                m0 = (lanes == 0) & valid
                tv = jnp.full((LANES,), t, jnp.int32)
                bx1 = plsc.load_gather(vbf, [ci4])
                by1 = plsc.load_gather(vbf, [ci4 + 1])
                bx2 = plsc.load_gather(vbf, [ci4 + 2])
                by2 = plsc.load_gather(vbf, [ci4 + 3])
                lbv = plsc.load_gather(vlb, [cis])
                plsc.store_scatter(rb, [tv * 4], bx1, mask=m0)
                plsc.store_scatter(rb, [tv * 4 + 1], by1, mask=m0)
                plsc.store_scatter(rb, [tv * 4 + 2], bx2, mask=m0)
                plsc.store_scatter(rb, [tv * 4 + 3], by2, mask=m0)
                plsc.store_scatter(rs, [tv], cs, mask=m0)
                plsc.store_scatter(rl, [tv], lbv, mask=m0)
                adv = valid.astype(jnp.int32)
                hp = hp + jnp.where(takeh, adv, 0)
                op = op + jnp.where(takeh, 0, adv)
                return (hp, op)

            lax.fori_loop(0, 2 * KCAP, mg_body, (jnp.int32(0), jnp.int32(0)))

            pltpu.sync_copy(rb, obh)
            pltpu.sync_copy(rs, osh)
            pltpu.sync_copy(rl, olh)


_mesh = plsc.VectorSubcoreMesh(core_axis_name="c", subcore_axis_name="s",
                               num_cores=2, num_subcores=16)

_OUT_TYPE = [
    jax.ShapeDtypeStruct((128,), jnp.float32),
    jax.ShapeDtypeStruct((32,), jnp.float32),
    jax.ShapeDtypeStruct((32,), jnp.int32),
]

_SCRATCH_TYPES = [
    pltpu.VMEM((4 * N,), jnp.float32),
    pltpu.VMEM((NPAD + LANES,), jnp.float32),
    pltpu.VMEM((NPAD + LANES,), jnp.int32),
    pltpu.VMEM((NPAD + LANES,), jnp.int32),
    pltpu.VMEM((NPAD + LANES,), jnp.int32),
    pltpu.VMEM((NPAD + LANES,), jnp.int32),
    pltpu.VMEM((NPAD + 2 * LANES,), jnp.float32),
    pltpu.VMEM((CPT * LANES,), jnp.float32),
    pltpu.VMEM((CPT * LANES,), jnp.int32),
    pltpu.VMEM((NCLS * LANES,), jnp.float32),
    pltpu.VMEM((NCLS * LANES,), jnp.int32),
    pltpu.VMEM((NCLS,), jnp.float32),
    pltpu.VMEM((NCLS,), jnp.int32),
    pltpu.VMEM((NCLS,), jnp.int32),
    pltpu.VMEM((128,), jnp.float32),
    pltpu.VMEM((32,), jnp.float32),
    pltpu.VMEM((32,), jnp.int32),
    pltpu.VMEM((TILES * LANES,), jnp.float32),
    pltpu.SemaphoreType.DMA,
    pltpu.SemaphoreType.DMA,
    pltpu.SemaphoreType.DMA,
    pltpu.VMEM_SHARED((NCLS * LANES,), jnp.float32),
    pltpu.VMEM_SHARED((NCLS * LANES,), jnp.int32),
    pltpu.VMEM_SHARED((TILES * LANES,), jnp.float32),
]

_sc_call = pl.kernel(
    _nms_body,
    out_type=_OUT_TYPE,
    mesh=_mesh,
    compiler_params=pltpu.CompilerParams(needs_layout_passes=False),
    scratch_types=_SCRATCH_TYPES,
)


@jax.jit
def kernel(boxes, scores, labels):
    obf, osf, olf = _sc_call(boxes.reshape(-1), scores, labels)
    return obf[:120].reshape(30, 4), osf[:30], olf[:30]


# speedup vs baseline: 1.0672x; 1.0002x over previous
import jax
import jax.numpy as jnp
from jax import lax
from jax.experimental import pallas as pl
from jax.experimental.pallas import tpu as pltpu
from jax.experimental.pallas import tpu_sc as plsc

N = 5000
LANES = 16
NPAD = 5120
NCH = NPAD // LANES
NCLS = 80
HUMAN_IDX = 1
NMS_THRESH = 0.5
SCORE_THRESH = 0.2
KCAP = 15
TILES = 16
CPT = NCLS // TILES
NEGS = -3.0e38
DUMMY = 3.0e9
BIGI = 2**30


def _nms_body(bfh, sch, lbh, obh, osh, olh,
              vbf, vsc, vlb, rmidx, rmlab, midx, ms,
              kvs, kvi, gsc, gidx, heads_s, heads_i, ptrv, rb, rs, rl, lmax,
              sem1, sem2, sem3, ssc, sidx, smax):
    core = lax.axis_index("c")
    sub = lax.axis_index("s")
    lanes = lax.iota(jnp.int32, LANES)
    ones = lanes >= 0
    negs16 = jnp.full((LANES,), NEGS, jnp.float32)
    bigi16 = jnp.full((LANES,), BIGI, jnp.int32)

    @pl.when(core == 0)
    def _():
        cp1 = pltpu.async_copy(bfh, vbf, sem1)
        cp2 = pltpu.async_copy(sch, vsc.at[pl.ds(0, N)], sem2)
        cp3 = pltpu.async_copy(lbh, vlb.at[pl.ds(0, N)], sem3)
        cp2.wait()
        cp3.wait()
        negone = jnp.full((LANES,), -1.0, jnp.float32)
        negonei = jnp.full((LANES,), -1, jnp.int32)
        for k in range(8):
            plsc.store_compressed(vsc.at[pl.ds(N + 16 * k, LANES)], negone,
                                  mask=ones)
            plsc.store_compressed(vlb.at[pl.ds(N + 16 * k, LANES)], negonei,
                                  mask=ones)
        vsc[pl.ds(NPAD, LANES)] = negone
        vlb[pl.ds(NPAD, LANES)] = negonei

        lo = sub * CPT

        def rchunk(j, cnt):
            lab16 = vlb[pl.ds(j * LANES, LANES)]
            sc16 = vsc[pl.ds(j * LANES, LANES)]
            m = (lab16 >= lo) & (lab16 < lo + CPT) & (sc16 >= SCORE_THRESH)
            idx16 = j * LANES + lanes
            plsc.store_compressed(rmidx.at[pl.ds(cnt, LANES)], idx16, mask=m)
            plsc.store_compressed(rmlab.at[pl.ds(cnt, LANES)], lab16, mask=m)
            return cnt + plsc.all_reduce_population_count(m)[0]

        def rscan(j, cnt):
            cnt = rchunk(2 * j, cnt)
            return rchunk(2 * j + 1, cnt)

        rcnt = lax.fori_loop(0, NCH // 2, rscan, jnp.int32(0))
        plsc.store_compressed(rmlab.at[pl.ds(rcnt, LANES)],
                              jnp.full((LANES,), -1, jnp.int32), mask=ones)
        rch = (rcnt + (LANES - 1)) >> 4

        MXCH = (4 * N) // LANES
        MPT = MXCH // TILES

        def mx_body(j, acc):
            base = (sub * MPT + j) * LANES
            return jnp.maximum(acc, vbf[pl.ds(base, LANES)])

        cp1.wait()
        acc = lax.fori_loop(0, MPT, mx_body, negs16)

        @pl.when(sub == 0)
        def _():
            a2 = jnp.maximum(vbf[pl.ds(MPT * TILES * LANES, LANES)],
                             vbf[pl.ds(MPT * TILES * LANES + LANES, LANES)])
            kvs[pl.ds(0, LANES)] = jnp.maximum(acc, a2)

        @pl.when(sub != 0)
        def _():
            kvs[pl.ds(0, LANES)] = acc

        pltpu.sync_copy(kvs.at[pl.ds(0, LANES)],
                        smax.at[pl.ds(sub * LANES, LANES)])
        plsc.subcore_barrier()
        pltpu.sync_copy(smax, lmax)
        macc = negs16
        for j in range(TILES):
            macc = jnp.maximum(macc, lmax[pl.ds(j * LANES, LANES)])
        maxc = jnp.max(macc) + jnp.float32(1.0)

        for k in range(CPT):
            c = lo + k
            off = c.astype(jnp.float32) * maxc

            def scan_body(j, cnt):
                lab16 = rmlab[pl.ds(j * LANES, LANES)]
                m = lab16 == c
                plsc.store_compressed(midx.at[pl.ds(cnt, LANES)],
                                      rmidx[pl.ds(j * LANES, LANES)], mask=m)
                return cnt + plsc.all_reduce_population_count(m)[0]

            cnt = lax.fori_loop(0, rch, scan_body, jnp.int32(0))
            nchk0 = (cnt + (LANES - 1)) >> 4

            def ms_body(j, _):
                mi = midx[pl.ds(j * LANES, LANES)]
                ms[pl.ds(j * LANES, LANES)] = plsc.load_gather(vsc, [mi])
                return 0

            plsc.store_compressed(midx.at[pl.ds(cnt, LANES)],
                                  jnp.zeros((LANES,), jnp.int32), mask=ones)
            lax.fori_loop(0, nchk0, ms_body, 0)
            plsc.store_compressed(ms.at[pl.ds(cnt, LANES)], negs16, mask=ones)
            plsc.store_compressed(ms.at[pl.ds(cnt + LANES, LANES)], negs16,
                                  mask=ones)

            def cond(st):
                return (st[0] < cnt) & (st[1] < KCAP)

            def body(st):
                nproc, kcnt, kx1, ky1, kx2, ky2, kid, ksc = st
                nchk2 = (cnt + (2 * LANES - 1)) >> 5

                def am_body(j, s):
                    bv, bp = s
                    v = ms[pl.ds(2 * j * LANES, LANES)]
                    upd = v > bv
                    bv = jnp.where(upd, v, bv)
                    bp = jnp.where(upd, 2 * j, bp)
                    v = ms[pl.ds((2 * j + 1) * LANES, LANES)]
                    upd = v > bv
                    return jnp.where(upd, v, bv), jnp.where(upd, 2 * j + 1, bp)

                bv, bp = lax.fori_loop(0, nchk2, am_body,
                                       (negs16, jnp.zeros((LANES,), jnp.int32)))
                gmax = jnp.max(bv)
                posl = jnp.where(bv == gmax, bp * LANES + lanes, BIGI)
                pos = jnp.min(posl)
                posv = jnp.full((LANES,), pos, jnp.int32)
                plsc.store_scatter(ms, [posv], negs16, mask=lanes == 0)
                giv = plsc.load_gather(midx, [posv])
                g4 = giv * 4
                cx1 = plsc.load_gather(vbf, [g4]) + off
                cy1 = plsc.load_gather(vbf, [g4 + 1]) + off
                cx2 = plsc.load_gather(vbf, [g4 + 2]) + off
                cy2 = plsc.load_gather(vbf, [g4 + 3]) + off
                w = jnp.maximum(jnp.minimum(kx2, cx2) - jnp.maximum(kx1, cx1), 0.0)
                h = jnp.maximum(jnp.minimum(ky2, cy2) - jnp.maximum(ky1, cy1), 0.0)
                inter = w * h
                ka = (kx2 - kx1) * (ky2 - ky1)
                ca = (cx2 - cx1) * (cy2 - cy1)
                iou = inter / jnp.maximum(ka + ca - inter, jnp.float32(1e-9))
                sup = plsc.all_reduce_population_count(iou > NMS_THRESH)[0] > 0
                addm = jnp.logical_and(jnp.logical_not(sup), lanes == kcnt)
                kx1 = jnp.where(addm, cx1, kx1)
                ky1 = jnp.where(addm, cy1, ky1)
                kx2 = jnp.where(addm, cx2, kx2)
                ky2 = jnp.where(addm, cy2, ky2)
                kid = jnp.where(addm, giv, kid)
                ksc = jnp.where(addm, gmax, ksc)
                kcnt = kcnt + jnp.where(sup, 0, 1).astype(jnp.int32)
                return (nproc + 1, kcnt, kx1, ky1, kx2, ky2, kid, ksc)

            dummy16 = jnp.full((LANES,), DUMMY, jnp.float32)
            st = lax.while_loop(cond, body,
                                (jnp.int32(0), jnp.int32(0),
                                 dummy16, dummy16, dummy16, dummy16,
                                 bigi16, negs16))
            kvs[pl.ds(k * LANES, LANES)] = st[7]
            kvi[pl.ds(k * LANES, LANES)] = st[6]

        pltpu.sync_copy(kvs, ssc.at[pl.ds(lo * LANES, CPT * LANES)])
        pltpu.sync_copy(kvi, sidx.at[pl.ds(lo * LANES, CPT * LANES)])

        plsc.subcore_barrier()

        @pl.when(sub == 0)
        def _():
            pltpu.sync_copy(ssc, gsc)
            pltpu.sync_copy(sidx, gidx)
            hs = gsc[pl.ds(HUMAN_IDX * LANES, LANES)]
            hi = gidx[pl.ds(HUMAN_IDX * LANES, LANES)]
            gsc[pl.ds(HUMAN_IDX * LANES, LANES)] = negs16
            for j in range(NCLS // LANES):
                rowv = (j * LANES + lanes) * LANES
                heads_s[pl.ds(j * LANES, LANES)] = plsc.load_gather(gsc, [rowv])
                heads_i[pl.ds(j * LANES, LANES)] = plsc.load_gather(gidx, [rowv])
            one16 = jnp.full((LANES,), 1, jnp.int32)
            for j in range(NCLS // LANES):
                ptrv[pl.ds(j * LANES, LANES)] = one16

            def ext_body(t, s):
                osc, oidx = s

                bv, bi, bp = (negs16, bigi16, jnp.zeros((LANES,), jnp.int32))
                for j in range(NCLS // LANES):
                    v = heads_s[pl.ds(j * LANES, LANES)]
                    iv = heads_i[pl.ds(j * LANES, LANES)]
                    upd = (v > bv) | ((v == bv) & (iv < bi))
                    bv, bi, bp = (jnp.where(upd, v, bv),
                                  jnp.where(upd, iv, bi),
                                  jnp.where(upd, j, bp))
                gmax = jnp.max(bv)
                gidw = jnp.min(jnp.where(bv == gmax, bi, BIGI))
                cls = jnp.min(jnp.where((bv == gmax) & (bi == gidw),
                                        bp * LANES + lanes, BIGI))
                clsv = jnp.full((LANES,), cls, jnp.int32)
                p = plsc.load_gather(ptrv, [clsv])
                plsc.store_scatter(ptrv, [clsv], p + 1, mask=lanes == 0)
                psafe = jnp.minimum(p, LANES - 1)
                hv = plsc.load_gather(gsc, [clsv * LANES + psafe])
                hiv = plsc.load_gather(gidx, [clsv * LANES + psafe])
                hv = jnp.where(p >= LANES, negs16, hv)
                plsc.store_scatter(heads_s, [clsv], hv, mask=lanes == 0)
                plsc.store_scatter(heads_i, [clsv], hiv, mask=lanes == 0)
                valid = gmax > jnp.float32(-1.0e37)
                osc = jnp.where((lanes == t) & valid, gmax, osc)
                oidx = jnp.where((lanes == t) & valid, gidw, oidx)
                return (osc, oidx)

            osc, oidx = lax.fori_loop(0, KCAP, ext_body, (negs16, bigi16))

            heads_s[pl.ds(0, LANES)] = hs
            heads_i[pl.ds(0, LANES)] = hi
            heads_s[pl.ds(LANES, LANES)] = osc
            heads_i[pl.ds(LANES, LANES)] = oidx

            zf16 = jnp.zeros((LANES,), jnp.float32)
            for j in range(8):
                rb[pl.ds(j * LANES, LANES)] = zf16
            rs[pl.ds(0, LANES)] = zf16
            rs[pl.ds(LANES, LANES)] = zf16
            neg1 = jnp.full((LANES,), -1, jnp.int32)
            rl[pl.ds(0, LANES)] = neg1
            rl[pl.ds(LANES, LANES)] = neg1

            def mg_body(t, s):
                hp, op = s
                hpv = jnp.full((LANES,), hp, jnp.int32)
                opv = jnp.full((LANES,), op + LANES, jnp.int32)
                hv = plsc.load_gather(heads_s, [hpv])
                hiv = plsc.load_gather(heads_i, [hpv])
                ov = plsc.load_gather(heads_s, [opv])
                oiv = plsc.load_gather(heads_i, [opv])
                hvs = hv[0]
                ovs = ov[0]
                his = hiv[0]
                ois = oiv[0]
                takeh = (hvs > ovs) | ((hvs == ovs) & (his < ois))
                cs = jnp.where(takeh, hv, ov)
                ci = jnp.where(takeh, hiv, oiv)
                valid = cs[0] > jnp.float32(-1.0e37)
                cis = jnp.where(valid, ci, jnp.zeros((LANES,), jnp.int32))
                ci4 = cis * 4
                m0 = (lanes == 0) & valid
                tv = jnp.full((LANES,), t, jnp.int32)
                bx1 = plsc.load_gather(vbf, [ci4])
                by1 = plsc.load_gather(vbf, [ci4 + 1])
                bx2 = plsc.load_gather(vbf, [ci4 + 2])
                by2 = plsc.load_gather(vbf, [ci4 + 3])
                lbv = plsc.load_gather(vlb, [cis])
                plsc.store_scatter(rb, [tv * 4], bx1, mask=m0)
                plsc.store_scatter(rb, [tv * 4 + 1], by1, mask=m0)
                plsc.store_scatter(rb, [tv * 4 + 2], bx2, mask=m0)
                plsc.store_scatter(rb, [tv * 4 + 3], by2, mask=m0)
                plsc.store_scatter(rs, [tv], cs, mask=m0)
                plsc.store_scatter(rl, [tv], lbv, mask=m0)
                adv = valid.astype(jnp.int32)
                hp = hp + jnp.where(takeh, adv, 0)
                op = op + jnp.where(takeh, 0, adv)
                return (hp, op)

            lax.fori_loop(0, 2 * KCAP, mg_body, (jnp.int32(0), jnp.int32(0)))

            pltpu.sync_copy(rb, obh)
            pltpu.sync_copy(rs, osh)
            pltpu.sync_copy(rl, olh)


_mesh = plsc.VectorSubcoreMesh(core_axis_name="c", subcore_axis_name="s",
                               num_cores=2, num_subcores=16)

_OUT_TYPE = [
    jax.ShapeDtypeStruct((128,), jnp.float32),
    jax.ShapeDtypeStruct((32,), jnp.float32),
    jax.ShapeDtypeStruct((32,), jnp.int32),
]

_SCRATCH_TYPES = [
    pltpu.VMEM((4 * N,), jnp.float32),
    pltpu.VMEM((NPAD + LANES,), jnp.float32),
    pltpu.VMEM((NPAD + LANES,), jnp.int32),
    pltpu.VMEM((NPAD + LANES,), jnp.int32),
    pltpu.VMEM((NPAD + LANES,), jnp.int32),
    pltpu.VMEM((NPAD + LANES,), jnp.int32),
    pltpu.VMEM((NPAD + 2 * LANES,), jnp.float32),
    pltpu.VMEM((CPT * LANES,), jnp.float32),
    pltpu.VMEM((CPT * LANES,), jnp.int32),
    pltpu.VMEM((NCLS * LANES,), jnp.float32),
    pltpu.VMEM((NCLS * LANES,), jnp.int32),
    pltpu.VMEM((NCLS,), jnp.float32),
    pltpu.VMEM((NCLS,), jnp.int32),
    pltpu.VMEM((NCLS,), jnp.int32),
    pltpu.VMEM((128,), jnp.float32),
    pltpu.VMEM((32,), jnp.float32),
    pltpu.VMEM((32,), jnp.int32),
    pltpu.VMEM((TILES * LANES,), jnp.float32),
    pltpu.SemaphoreType.DMA,
    pltpu.SemaphoreType.DMA,
    pltpu.SemaphoreType.DMA,
    pltpu.VMEM_SHARED((NCLS * LANES,), jnp.float32),
    pltpu.VMEM_SHARED((NCLS * LANES,), jnp.int32),
    pltpu.VMEM_SHARED((TILES * LANES,), jnp.float32),
]

_sc_call = pl.kernel(
    _nms_body,
    out_type=_OUT_TYPE,
    mesh=_mesh,
    compiler_params=pltpu.CompilerParams(needs_layout_passes=False),
    scratch_types=_SCRATCH_TYPES,
)


@jax.jit
def kernel(boxes, scores, labels):
    obf, osf, olf = _sc_call(boxes.reshape(-1), scores, labels)
    return obf[:120].reshape(30, 4), osf[:30], olf[:30]
